# Initial kernel scaffold; baseline (speedup 1.0000x reference)
import jax
import jax.numpy as jnp
from jax.experimental import pallas as pl


def kernel(u_embedded, i_embedded, user_per_trans, item_per_trans, edges_t, u_t, i_t, wu_w1, wu_b1, wu_g1, wu_be1, wu_w2, wu_b2, wi_w1, wi_b1, wi_g1, wi_be1, wi_w2, wi_b2):
    raise NotImplementedError("write your pallas kernel here")


# speedup vs baseline: 1.0782x; 1.0782x over previous
import functools

import jax
import jax.numpy as jnp
from jax.experimental import pallas as pl
from jax.experimental.pallas import tpu as pltpu

HID = 64
KH = 50
OMEGA = 30.0


def _dense_body(rel_ref, emb_ref, w1_ref, b1_ref, g1_ref, be1_ref, w2t_ref,
                b2_ref, out_ref):
    x = rel_ref[:]
    h1 = x * w1_ref[:] + b1_ref[:]
    mu = jnp.mean(h1, axis=1, keepdims=True)
    d = h1 - mu
    var = jnp.mean(d * d, axis=1, keepdims=True)
    h = d * jax.lax.rsqrt(var + 1e-5) * g1_ref[:] + be1_ref[:]
    h = jnp.sin(OMEGA * h)
    k = jnp.dot(h, w2t_ref[:], preferred_element_type=jnp.float32) + b2_ref[:]
    eb = k.shape[0]
    k3 = k.reshape(eb, HID, HID)
    emb = emb_ref[:]
    out_ref[:] = jnp.sum(k3 * emb[:, None, :], axis=2)


def _dense_side(rel, emb_g, w1, b1, g1, be1, w2, b2, eb, interpret=False):
    e_pad = rel.shape[0]
    grid = (e_pad // eb,)
    return pl.pallas_call(
        _dense_body,
        grid=grid,
        in_specs=[
            pl.BlockSpec((eb, 1), lambda i: (i, 0)),
            pl.BlockSpec((eb, HID), lambda i: (i, 0)),
            pl.BlockSpec((1, KH), lambda i: (0, 0)),
            pl.BlockSpec((1, KH), lambda i: (0, 0)),
            pl.BlockSpec((1, KH), lambda i: (0, 0)),
            pl.BlockSpec((1, KH), lambda i: (0, 0)),
            pl.BlockSpec((KH, HID * HID), lambda i: (0, 0)),
            pl.BlockSpec((1, HID * HID), lambda i: (0, 0)),
        ],
        out_specs=pl.BlockSpec((eb, HID), lambda i: (i, 0)),
        out_shape=jax.ShapeDtypeStruct((e_pad, HID), jnp.float32),
        interpret=interpret,
    )(rel, emb_g, w1.reshape(1, KH), b1.reshape(1, KH), g1.reshape(1, KH),
      be1.reshape(1, KH), w2.T, b2.reshape(1, HID * HID))


def kernel(u_embedded, i_embedded, user_per_trans, item_per_trans, edges_t,
           u_t, i_t,
           wu_w1, wu_b1, wu_g1, wu_be1, wu_w2, wu_b2,
           wi_w1, wi_b1, wi_g1, wi_be1, wi_w2, wi_b2,
           interpret=False):
    e = edges_t.shape[0]
    eb = 512
    e_pad = ((e + eb - 1) // eb) * eb
    pad = e_pad - e

    up = jnp.pad(user_per_trans, (0, pad))
    ip = jnp.pad(item_per_trans, (0, pad))
    et = jnp.pad(edges_t, (0, pad))

    rel_u = (u_t[up] - et).reshape(e_pad, 1)
    rel_i = (i_t[ip] - et).reshape(e_pad, 1)
    embg_u = u_embedded[up]
    embg_i = i_embedded[ip]

    msg_item = _dense_side(rel_i, embg_i, wi_w1, wi_b1, wi_g1, wi_be1, wi_w2,
                           wi_b2, eb, interpret)
    msg_user = _dense_side(rel_u, embg_u, wu_w1, wu_b1, wu_g1, wu_be1, wu_w2,
                           wu_b2, eb, interpret)

    msg_item = msg_item[:e]
    msg_user = msg_user[:e]
    hlu = jnp.zeros_like(u_embedded).at[user_per_trans].add(msg_item)
    hli = jnp.zeros_like(i_embedded).at[item_per_trans].add(msg_user)
    return (hlu, hli)


# speedup vs baseline: 2.1537x; 1.9974x over previous
import functools

import jax
import jax.numpy as jnp
from jax.experimental import pallas as pl
from jax.experimental.pallas import tpu as pltpu

HID = 64
KH = 50
OMEGA = 30.0


def _dense_body(rel_ref, embt_ref, w1_ref, b1_ref, g1_ref, be1_ref, wall_ref,
                out_ref):
    x = rel_ref[0]
    h1 = w1_ref[:] * x + b1_ref[:]
    mu = jnp.mean(h1, axis=0, keepdims=True)
    d = h1 - mu
    var = jnp.mean(d * d, axis=0, keepdims=True)
    h = d * jax.lax.rsqrt(var + 1e-5) * g1_ref[:] + be1_ref[:]
    h = jnp.sin(OMEGA * h)
    embt = embt_ref[:]
    eb = embt.shape[1]
    hexp = jnp.broadcast_to(h[:, None, :], (KH, HID, eb))
    ot = (hexp * embt[None, :, :]).reshape(KH * HID, eb)
    ofull = jnp.concatenate([ot, embt], axis=0)
    msgt = jnp.dot(wall_ref[:], ofull, preferred_element_type=jnp.float32)
    out_ref[:] = msgt.T


def _dense_side(rel3, embt, w1, b1, g1, be1, w2, b2, eb, interpret=False):
    nb = rel3.shape[0]
    e_pad = nb * eb
    kfull = KH * HID + HID
    w2r = w2.reshape(HID, HID, KH)
    wall = jnp.concatenate(
        [w2r.transpose(0, 2, 1).reshape(HID, KH * HID),
         b2.reshape(HID, HID)], axis=1)
    return pl.pallas_call(
        _dense_body,
        grid=(nb,),
        in_specs=[
            pl.BlockSpec((1, 1, eb), lambda i: (i, 0, 0)),
            pl.BlockSpec((HID, eb), lambda i: (0, i)),
            pl.BlockSpec((KH, 1), lambda i: (0, 0)),
            pl.BlockSpec((KH, 1), lambda i: (0, 0)),
            pl.BlockSpec((KH, 1), lambda i: (0, 0)),
            pl.BlockSpec((KH, 1), lambda i: (0, 0)),
            pl.BlockSpec((HID, kfull), lambda i: (0, 0)),
        ],
        out_specs=pl.BlockSpec((eb, HID), lambda i: (i, 0)),
        out_shape=jax.ShapeDtypeStruct((e_pad, HID), jnp.float32),
        interpret=interpret,
    )(rel3, embt, w1.reshape(KH, 1), b1.reshape(KH, 1), g1.reshape(KH, 1),
      be1.reshape(KH, 1), wall)


def kernel(u_embedded, i_embedded, user_per_trans, item_per_trans, edges_t,
           u_t, i_t,
           wu_w1, wu_b1, wu_g1, wu_be1, wu_w2, wu_b2,
           wi_w1, wi_b1, wi_g1, wi_be1, wi_w2, wi_b2,
           interpret=False):
    e = edges_t.shape[0]
    eb = 512
    e_pad = ((e + eb - 1) // eb) * eb
    pad = e_pad - e
    nb = e_pad // eb

    up = jnp.pad(user_per_trans, (0, pad))
    ip = jnp.pad(item_per_trans, (0, pad))
    et = jnp.pad(edges_t, (0, pad))

    rel_u = (u_t[up] - et).reshape(nb, 1, eb)
    rel_i = (i_t[ip] - et).reshape(nb, 1, eb)
    embt_u = u_embedded[up].T
    embt_i = i_embedded[ip].T

    msg_item = _dense_side(rel_i, embt_i, wi_w1, wi_b1, wi_g1, wi_be1, wi_w2,
                           wi_b2, eb, interpret)
    msg_user = _dense_side(rel_u, embt_u, wu_w1, wu_b1, wu_g1, wu_be1, wu_w2,
                           wu_b2, eb, interpret)

    msg_item = msg_item[:e]
    msg_user = msg_user[:e]
    hlu = jnp.zeros_like(u_embedded).at[user_per_trans].add(msg_item)
    hli = jnp.zeros_like(i_embedded).at[item_per_trans].add(msg_user)
    return (hlu, hli)


# speedup vs baseline: 2.5029x; 1.1622x over previous
import functools

import jax
import jax.numpy as jnp
from jax import lax
from jax.experimental import pallas as pl
from jax.experimental.pallas import tpu as pltpu
from jax.experimental.pallas import tpu_sc as plsc

HID = 64
KH = 50
OMEGA = 30.0

_NC = 2
_NS = 16
_GRP = 128


def _dense_body(rel_ref, embt_ref, w1_ref, b1_ref, g1_ref, be1_ref, wall_ref,
                out_ref):
    x = rel_ref[0]
    h1 = w1_ref[:] * x + b1_ref[:]
    mu = jnp.mean(h1, axis=0, keepdims=True)
    d = h1 - mu
    var = jnp.mean(d * d, axis=0, keepdims=True)
    h = d * jax.lax.rsqrt(var + 1e-5) * g1_ref[:] + be1_ref[:]
    h = jnp.sin(OMEGA * h)
    embt = embt_ref[:]
    eb = embt.shape[1]
    hexp = jnp.broadcast_to(h[:, None, :], (KH, HID, eb))
    ot = (hexp * embt[None, :, :]).reshape(KH * HID, eb)
    ofull = jnp.concatenate([ot, embt], axis=0)
    msgt = jnp.dot(wall_ref[:], ofull, preferred_element_type=jnp.float32)
    out_ref[:] = msgt.T


def _dense_side(rel3, embt, w1, b1, g1, be1, w2, b2, eb):
    nb = rel3.shape[0]
    e_pad = nb * eb
    kfull = KH * HID + HID
    w2r = w2.reshape(HID, HID, KH)
    wall = jnp.concatenate(
        [w2r.transpose(0, 2, 1).reshape(HID, KH * HID),
         b2.reshape(HID, HID)], axis=1)
    return pl.pallas_call(
        _dense_body,
        grid=(nb,),
        in_specs=[
            pl.BlockSpec((1, 1, eb), lambda i: (i, 0, 0)),
            pl.BlockSpec((HID, eb), lambda i: (0, i)),
            pl.BlockSpec((KH, 1), lambda i: (0, 0)),
            pl.BlockSpec((KH, 1), lambda i: (0, 0)),
            pl.BlockSpec((KH, 1), lambda i: (0, 0)),
            pl.BlockSpec((KH, 1), lambda i: (0, 0)),
            pl.BlockSpec((HID, kfull), lambda i: (0, 0)),
        ],
        out_specs=pl.BlockSpec((eb, HID), lambda i: (i, 0)),
        out_shape=jax.ShapeDtypeStruct((e_pad, HID), jnp.float32),
    )(rel3, embt, w1.reshape(KH, 1), b1.reshape(KH, 1), g1.reshape(KH, 1),
      be1.reshape(KH, 1), wall)


def _scatter_add(msg, idx, zeros, n_rows):
    e_pad = msg.shape[0]
    assert e_pad % (_NS * _GRP) == 0
    chunk = e_pad // _NS
    ngrp = chunk // _GRP
    half = n_rows // 2
    rpt = -(-(-(-half // _NS)) // 8) * 8
    last = half - (_NS - 1) * rpt
    assert last > 0 and last % 8 == 0 and rpt <= zeros.shape[0]
    mesh = plsc.VectorSubcoreMesh(core_axis_name="c", subcore_axis_name="s")

    @functools.partial(
        pl.kernel, mesh=mesh,
        out_type=jax.ShapeDtypeStruct((n_rows, HID), jnp.float32),
        compiler_params=pltpu.CompilerParams(use_tc_tiling_on_sc=False),
        scratch_types=[
            pltpu.VMEM((chunk,), jnp.int32),
            pltpu.VMEM((ngrp, _GRP), jnp.int32),
            pltpu.VMEM((2, _GRP, HID), jnp.float32),
            pltpu.VMEM_SHARED((half + 8, HID), jnp.float32),
            pltpu.SemaphoreType.DMA,
            pltpu.SemaphoreType.DMA,
        ],
    )
    def k(msg_hbm, idx_hbm, zeros_hbm, out_hbm, idx_v, lidx_v, msg_v, acc_sh,
          sem0, sem1):
        c = lax.axis_index("c")
        s = lax.axis_index("s")
        half_i = jnp.int32(half)
        sems = [sem0, sem1]

        @pl.when(s < _NS - 1)
        def _():
            pltpu.sync_copy(zeros_hbm.at[pl.ds(0, rpt)],
                            acc_sh.at[pl.ds(s * rpt, rpt)])

        @pl.when(s == _NS - 1)
        def _():
            pltpu.sync_copy(zeros_hbm.at[pl.ds(0, last)],
                            acc_sh.at[pl.ds(s * rpt, last)])

        base = s * chunk
        pltpu.sync_copy(idx_hbm.at[pl.ds(base, chunk)], idx_v)
        for g in range(ngrp):
            for l in range(_GRP // 16):
                o = g * _GRP + l * 16
                v = idx_v[pl.ds(o, 16)] - c * half_i
                ok = (v >= 0) & (v < half_i)
                lidx_v[g, pl.ds(l * 16, 16)] = jnp.where(ok, v, half_i)

        plsc.subcore_barrier()

        loads = [None, None]
        loads[0] = pltpu.async_copy(
            msg_hbm.at[pl.ds(base, _GRP)], msg_v.at[0], sems[0])
        for g in range(ngrp):
            b = g % 2
            if g + 1 < ngrp:
                loads[1 - b] = pltpu.async_copy(
                    msg_hbm.at[pl.ds(base + (g + 1) * _GRP, _GRP)],
                    msg_v.at[1 - b], sems[1 - b])
            loads[b].wait()
            pltpu.sync_copy(msg_v.at[b], acc_sh.at[lidx_v.at[g]], add=True)

        plsc.subcore_barrier()

        @pl.when(s < _NS - 1)
        def _():
            pltpu.sync_copy(acc_sh.at[pl.ds(s * rpt, rpt)],
                            out_hbm.at[pl.ds(c * half + s * rpt, rpt)])

        @pl.when(s == _NS - 1)
        def _():
            pltpu.sync_copy(acc_sh.at[pl.ds(s * rpt, last)],
                            out_hbm.at[pl.ds(c * half + s * rpt, last)])

    return k(msg, idx, zeros)


def kernel(u_embedded, i_embedded, user_per_trans, item_per_trans, edges_t,
           u_t, i_t,
           wu_w1, wu_b1, wu_g1, wu_be1, wu_w2, wu_b2,
           wi_w1, wi_b1, wi_g1, wi_be1, wi_w2, wi_b2):
    e = edges_t.shape[0]
    n_users = u_embedded.shape[0]
    n_items = i_embedded.shape[0]
    eb = 512
    quantum = _NS * _GRP
    e_pad = ((e + quantum - 1) // quantum) * quantum
    pad = e_pad - e
    nb = e_pad // eb

    up = jnp.pad(user_per_trans, (0, pad), constant_values=n_users)
    ip = jnp.pad(item_per_trans, (0, pad), constant_values=n_items)
    et = jnp.pad(edges_t, (0, pad))

    rel_u = (u_t[up] - et).reshape(nb, 1, eb)
    rel_i = (i_t[ip] - et).reshape(nb, 1, eb)
    embt_u = u_embedded[up].T
    embt_i = i_embedded[ip].T

    msg_item = _dense_side(rel_i, embt_i, wi_w1, wi_b1, wi_g1, wi_be1, wi_w2,
                           wi_b2, eb)
    msg_user = _dense_side(rel_u, embt_u, wu_w1, wu_b1, wu_g1, wu_be1, wu_w2,
                           wu_b2, eb)

    zeros = jnp.zeros((-(-max(n_users, n_items) // (2 * _NS * 8)) * 8, HID),
                      jnp.float32)
    hlu = _scatter_add(msg_item, up, zeros, n_users)
    hli = _scatter_add(msg_user, ip, zeros, n_items)
    return (hlu, hli)


# speedup vs baseline: 3.9657x; 1.5845x over previous
import functools

import jax
import jax.numpy as jnp
from jax import lax
from jax.experimental import pallas as pl
from jax.experimental.pallas import tpu as pltpu
from jax.experimental.pallas import tpu_sc as plsc

HID = 64
KH = 50
OMEGA = 30.0

_NC = 2
_NS = 16
_GRP = 128


def _dense_body(rel_ref, emb_ref, w1_ref, b1_ref, g1_ref, be1_ref, wall_ref,
                out_ref):
    x = rel_ref[0]
    h1 = w1_ref[:] * x + b1_ref[:]
    mu = jnp.mean(h1, axis=0, keepdims=True)
    d = h1 - mu
    var = jnp.mean(d * d, axis=0, keepdims=True)
    h = d * jax.lax.rsqrt(var + 1e-5) * g1_ref[:] + be1_ref[:]
    h = jnp.sin(OMEGA * h)
    embt = emb_ref[:].T
    eb = embt.shape[1]
    hexp = jnp.broadcast_to(h[:, None, :], (KH, HID, eb))
    ot = (hexp * embt[None, :, :]).reshape(KH * HID, eb)
    ofull = jnp.concatenate([ot, embt], axis=0)
    msgt = jnp.dot(wall_ref[:], ofull, preferred_element_type=jnp.float32)
    out_ref[:] = msgt.T


def _dense_side(rel3, embg, w1, b1, g1, be1, w2, b2, eb):
    nb = rel3.shape[0]
    e_pad = nb * eb
    kfull = KH * HID + HID
    w2r = w2.reshape(HID, HID, KH)
    wall = jnp.concatenate(
        [w2r.transpose(0, 2, 1).reshape(HID, KH * HID),
         b2.reshape(HID, HID)], axis=1)
    return pl.pallas_call(
        _dense_body,
        grid=(nb,),
        in_specs=[
            pl.BlockSpec((1, 1, eb), lambda i: (i, 0, 0)),
            pl.BlockSpec((eb, HID), lambda i: (i, 0)),
            pl.BlockSpec((KH, 1), lambda i: (0, 0)),
            pl.BlockSpec((KH, 1), lambda i: (0, 0)),
            pl.BlockSpec((KH, 1), lambda i: (0, 0)),
            pl.BlockSpec((KH, 1), lambda i: (0, 0)),
            pl.BlockSpec((HID, kfull), lambda i: (0, 0)),
        ],
        out_specs=pl.BlockSpec((eb, HID), lambda i: (i, 0)),
        out_shape=jax.ShapeDtypeStruct((e_pad, HID), jnp.float32),
    )(rel3, embg, w1.reshape(KH, 1), b1.reshape(KH, 1), g1.reshape(KH, 1),
      be1.reshape(KH, 1), wall)


def _gather_all(u_emb, i_emb, u_t, i_t, up, ip, et):
    e_pad = up.shape[0]
    nw = _NC * _NS
    chunk = e_pad // nw
    ngrp = chunk // _GRP
    nu = u_emb.shape[0]
    ni = i_emb.shape[0]
    mesh = plsc.VectorSubcoreMesh(core_axis_name="c", subcore_axis_name="s")

    @functools.partial(
        pl.kernel, mesh=mesh,
        out_type=(jax.ShapeDtypeStruct((e_pad, HID), jnp.float32),
                  jax.ShapeDtypeStruct((e_pad, HID), jnp.float32),
                  jax.ShapeDtypeStruct((e_pad,), jnp.float32),
                  jax.ShapeDtypeStruct((e_pad,), jnp.float32)),
        compiler_params=pltpu.CompilerParams(use_tc_tiling_on_sc=False),
        scratch_types=[
            pltpu.VMEM((chunk,), jnp.int32),
            pltpu.VMEM((ngrp, _GRP), jnp.int32),
            pltpu.VMEM((ngrp, _GRP), jnp.int32),
            pltpu.VMEM((chunk, HID), jnp.float32),
            pltpu.VMEM((chunk, HID), jnp.float32),
            pltpu.VMEM((chunk,), jnp.float32),
            pltpu.VMEM((chunk,), jnp.float32),
            pltpu.VMEM((chunk,), jnp.float32),
            pltpu.SemaphoreType.DMA,
            pltpu.SemaphoreType.DMA,
        ],
    )
    def k(u_emb_h, i_emb_h, u_t_h, i_t_h, up_h, ip_h, et_h,
          eu_h, ei_h, ru_h, ri_h,
          idx_v, cu_v, ci_v, rowsu_v, rowsi_v, tg_v, et_v, rel_v,
          sem_a, sem_b):
        c = lax.axis_index("c")
        s = lax.axis_index("s")
        base = (c * _NS + s) * chunk
        pltpu.sync_copy(et_h.at[pl.ds(base, chunk)], et_v)

        for side, (idx_h, cl_v, rows_v, emb_h, t_h, nn, eo_h, ro_h) in enumerate([
                (up_h, cu_v, rowsu_v, u_emb_h, u_t_h, nu, eu_h, ru_h),
                (ip_h, ci_v, rowsi_v, i_emb_h, i_t_h, ni, ei_h, ri_h)]):
            pltpu.sync_copy(idx_h.at[pl.ds(base, chunk)], idx_v)
            for g in range(ngrp):
                for l in range(_GRP // 16):
                    o = g * _GRP + l * 16
                    cl_v[g, pl.ds(l * 16, 16)] = jnp.minimum(
                        idx_v[pl.ds(o, 16)], jnp.int32(nn - 1))
            rowc = [pltpu.async_copy(emb_h.at[cl_v.at[g]],
                                     rows_v.at[pl.ds(g * _GRP, _GRP)], sem_a)
                    for g in range(ngrp)]
            tc = [pltpu.async_copy(t_h.at[cl_v.at[g]],
                                   tg_v.at[pl.ds(g * _GRP, _GRP)], sem_b)
                  for g in range(ngrp)]
            for h in tc:
                h.wait()
            for g in range(ngrp):
                for l in range(_GRP // 16):
                    o = g * _GRP + l * 16
                    rel_v[pl.ds(o, 16)] = (tg_v[pl.ds(o, 16)]
                                           - et_v[pl.ds(o, 16)])
            pltpu.sync_copy(rel_v, ro_h.at[pl.ds(base, chunk)])
            for h in rowc:
                h.wait()
            pltpu.sync_copy(rows_v, eo_h.at[pl.ds(base, chunk)])

    return k(u_emb, i_emb, u_t, i_t, up, ip, et)


def _scatter_add(msg, idx, zeros, n_rows):
    e_pad = msg.shape[0]
    assert e_pad % (_NS * _GRP) == 0
    chunk = e_pad // _NS
    ngrp = chunk // _GRP
    half = n_rows // 2
    rpt = -(-(-(-half // _NS)) // 8) * 8
    last = half - (_NS - 1) * rpt
    assert last > 0 and last % 8 == 0 and rpt <= zeros.shape[0]
    mesh = plsc.VectorSubcoreMesh(core_axis_name="c", subcore_axis_name="s")

    @functools.partial(
        pl.kernel, mesh=mesh,
        out_type=jax.ShapeDtypeStruct((n_rows, HID), jnp.float32),
        compiler_params=pltpu.CompilerParams(use_tc_tiling_on_sc=False),
        scratch_types=[
            pltpu.VMEM((chunk,), jnp.int32),
            pltpu.VMEM((ngrp, _GRP), jnp.int32),
            pltpu.VMEM((2, _GRP, HID), jnp.float32),
            pltpu.VMEM_SHARED((half + 8, HID), jnp.float32),
            pltpu.SemaphoreType.DMA,
            pltpu.SemaphoreType.DMA,
        ],
    )
    def k(msg_hbm, idx_hbm, zeros_hbm, out_hbm, idx_v, lidx_v, msg_v, acc_sh,
          sem0, sem1):
        c = lax.axis_index("c")
        s = lax.axis_index("s")
        half_i = jnp.int32(half)
        sems = [sem0, sem1]

        @pl.when(s < _NS - 1)
        def _():
            pltpu.sync_copy(zeros_hbm.at[pl.ds(0, rpt)],
                            acc_sh.at[pl.ds(s * rpt, rpt)])

        @pl.when(s == _NS - 1)
        def _():
            pltpu.sync_copy(zeros_hbm.at[pl.ds(0, last)],
                            acc_sh.at[pl.ds(s * rpt, last)])

        base = s * chunk
        pltpu.sync_copy(idx_hbm.at[pl.ds(base, chunk)], idx_v)
        for g in range(ngrp):
            for l in range(_GRP // 16):
                o = g * _GRP + l * 16
                v = idx_v[pl.ds(o, 16)] - c * half_i
                ok = (v >= 0) & (v < half_i)
                lidx_v[g, pl.ds(l * 16, 16)] = jnp.where(ok, v, half_i)

        plsc.subcore_barrier()

        loads = [None, None]
        loads[0] = pltpu.async_copy(
            msg_hbm.at[pl.ds(base, _GRP)], msg_v.at[0], sems[0])
        for g in range(ngrp):
            b = g % 2
            if g + 1 < ngrp:
                loads[1 - b] = pltpu.async_copy(
                    msg_hbm.at[pl.ds(base + (g + 1) * _GRP, _GRP)],
                    msg_v.at[1 - b], sems[1 - b])
            loads[b].wait()
            pltpu.sync_copy(msg_v.at[b], acc_sh.at[lidx_v.at[g]], add=True)

        plsc.subcore_barrier()

        @pl.when(s < _NS - 1)
        def _():
            pltpu.sync_copy(acc_sh.at[pl.ds(s * rpt, rpt)],
                            out_hbm.at[pl.ds(c * half + s * rpt, rpt)])

        @pl.when(s == _NS - 1)
        def _():
            pltpu.sync_copy(acc_sh.at[pl.ds(s * rpt, last)],
                            out_hbm.at[pl.ds(c * half + s * rpt, last)])

    return k(msg, idx, zeros)


def kernel(u_embedded, i_embedded, user_per_trans, item_per_trans, edges_t,
           u_t, i_t,
           wu_w1, wu_b1, wu_g1, wu_be1, wu_w2, wu_b2,
           wi_w1, wi_b1, wi_g1, wi_be1, wi_w2, wi_b2):
    e = edges_t.shape[0]
    n_users = u_embedded.shape[0]
    n_items = i_embedded.shape[0]
    eb = 512
    quantum = _NS * _GRP
    e_pad = ((e + quantum - 1) // quantum) * quantum
    pad = e_pad - e
    nb = e_pad // eb

    up = jnp.pad(user_per_trans, (0, pad), constant_values=n_users)
    ip = jnp.pad(item_per_trans, (0, pad), constant_values=n_items)
    et = jnp.pad(edges_t, (0, pad))

    embg_u, embg_i, rel_u_f, rel_i_f = _gather_all(
        u_embedded, i_embedded, u_t, i_t, up, ip, et)
    rel_u = rel_u_f.reshape(nb, 1, eb)
    rel_i = rel_i_f.reshape(nb, 1, eb)

    msg_item = _dense_side(rel_i, embg_i, wi_w1, wi_b1, wi_g1, wi_be1, wi_w2,
                           wi_b2, eb)
    msg_user = _dense_side(rel_u, embg_u, wu_w1, wu_b1, wu_g1, wu_be1, wu_w2,
                           wu_b2, eb)

    zeros = jnp.zeros((-(-max(n_users, n_items) // (2 * _NS * 8)) * 8, HID),
                      jnp.float32)
    hlu = _scatter_add(msg_item, up, zeros, n_users)
    hli = _scatter_add(msg_user, ip, zeros, n_items)
    return (hlu, hli)


# speedup vs baseline: 4.3757x; 1.1034x over previous
import functools

import jax
import jax.numpy as jnp
from jax import lax
from jax.experimental import pallas as pl
from jax.experimental.pallas import tpu as pltpu
from jax.experimental.pallas import tpu_sc as plsc

HID = 64
KH = 50
OMEGA = 30.0

_NC = 2
_NS = 16
_GRP = 128


def _dense_body(rel_ref, emb_ref, w1_ref, b1_ref, g1_ref, be1_ref, wall_ref,
                out_ref):
    x = rel_ref[0]
    h1 = w1_ref[:] * x + b1_ref[:]
    mu = jnp.mean(h1, axis=0, keepdims=True)
    d = h1 - mu
    var = jnp.mean(d * d, axis=0, keepdims=True)
    h = d * jax.lax.rsqrt(var + 1e-5) * g1_ref[:] + be1_ref[:]
    h = jnp.sin(OMEGA * h)
    embt = emb_ref[:].T
    eb = embt.shape[1]
    hexp = jnp.broadcast_to(h[:, None, :], (KH, HID, eb))
    ot = (hexp * embt[None, :, :]).reshape(KH * HID, eb)
    ofull = jnp.concatenate([ot, embt], axis=0)
    msgt = jnp.dot(wall_ref[:], ofull, preferred_element_type=jnp.float32)
    out_ref[:] = msgt.T


def _dense_side(rel3, embg, w1, b1, g1, be1, w2, b2, eb):
    nb = rel3.shape[0]
    e_pad = nb * eb
    kfull = KH * HID + HID
    w2r = w2.reshape(HID, HID, KH)
    wall = jnp.concatenate(
        [w2r.transpose(0, 2, 1).reshape(HID, KH * HID),
         b2.reshape(HID, HID)], axis=1)
    return pl.pallas_call(
        _dense_body,
        grid=(nb,),
        in_specs=[
            pl.BlockSpec((1, 1, eb), lambda i: (i, 0, 0)),
            pl.BlockSpec((eb, HID), lambda i: (i, 0)),
            pl.BlockSpec((KH, 1), lambda i: (0, 0)),
            pl.BlockSpec((KH, 1), lambda i: (0, 0)),
            pl.BlockSpec((KH, 1), lambda i: (0, 0)),
            pl.BlockSpec((KH, 1), lambda i: (0, 0)),
            pl.BlockSpec((HID, kfull), lambda i: (0, 0)),
        ],
        out_specs=pl.BlockSpec((eb, HID), lambda i: (i, 0)),
        out_shape=jax.ShapeDtypeStruct((e_pad, HID), jnp.float32),
    )(rel3, embg, w1.reshape(KH, 1), b1.reshape(KH, 1), g1.reshape(KH, 1),
      be1.reshape(KH, 1), wall)


def _gather_side(emb, t, idx, et):
    e_pad = idx.shape[0]
    nw = _NC * _NS
    chunk = e_pad // nw
    ngrp = chunk // _GRP
    nn = emb.shape[0]
    mesh = plsc.VectorSubcoreMesh(core_axis_name="c", subcore_axis_name="s")

    @functools.partial(
        pl.kernel, mesh=mesh,
        out_type=(jax.ShapeDtypeStruct((e_pad, HID), jnp.float32),
                  jax.ShapeDtypeStruct((e_pad,), jnp.float32)),
        compiler_params=pltpu.CompilerParams(use_tc_tiling_on_sc=False),
        scratch_types=[
            pltpu.VMEM((chunk,), jnp.int32),
            pltpu.VMEM((ngrp, _GRP), jnp.int32),
            pltpu.VMEM((chunk, HID), jnp.float32),
            pltpu.VMEM((chunk,), jnp.float32),
            pltpu.VMEM((chunk,), jnp.float32),
            pltpu.VMEM((chunk,), jnp.float32),
            pltpu.SemaphoreType.DMA,
            pltpu.SemaphoreType.DMA,
        ],
    )
    def k(emb_h, t_h, idx_h, et_h, eo_h, ro_h,
          idx_v, cl_v, rows_v, tg_v, et_v, rel_v, sem_a, sem_b):
        c = lax.axis_index("c")
        s = lax.axis_index("s")
        base = (c * _NS + s) * chunk
        pltpu.sync_copy(et_h.at[pl.ds(base, chunk)], et_v)
        pltpu.sync_copy(idx_h.at[pl.ds(base, chunk)], idx_v)
        for g in range(ngrp):
            for l in range(_GRP // 16):
                o = g * _GRP + l * 16
                cl_v[g, pl.ds(l * 16, 16)] = jnp.minimum(
                    idx_v[pl.ds(o, 16)], jnp.int32(nn - 1))
        rowc = [pltpu.async_copy(emb_h.at[cl_v.at[g]],
                                 rows_v.at[pl.ds(g * _GRP, _GRP)], sem_a)
                for g in range(ngrp)]
        tc = [pltpu.async_copy(t_h.at[cl_v.at[g]],
                               tg_v.at[pl.ds(g * _GRP, _GRP)], sem_b)
              for g in range(ngrp)]
        for h in tc:
            h.wait()
        for g in range(ngrp):
            for l in range(_GRP // 16):
                o = g * _GRP + l * 16
                rel_v[pl.ds(o, 16)] = tg_v[pl.ds(o, 16)] - et_v[pl.ds(o, 16)]
        pltpu.sync_copy(rel_v, ro_h.at[pl.ds(base, chunk)])
        for h in rowc:
            h.wait()
        pltpu.sync_copy(rows_v, eo_h.at[pl.ds(base, chunk)])

    return k(emb, t, idx, et)


def _scatter_add(msg, idx, zeros, n_rows):
    e_pad = msg.shape[0]
    assert e_pad % (_NS * _GRP) == 0
    chunk = e_pad // _NS
    ngrp = chunk // _GRP
    half = n_rows // 2
    rpt = -(-(-(-half // _NS)) // 8) * 8
    last = half - (_NS - 1) * rpt
    assert last > 0 and last % 8 == 0 and rpt <= zeros.shape[0]
    mesh = plsc.VectorSubcoreMesh(core_axis_name="c", subcore_axis_name="s")

    @functools.partial(
        pl.kernel, mesh=mesh,
        out_type=jax.ShapeDtypeStruct((n_rows, HID), jnp.float32),
        compiler_params=pltpu.CompilerParams(use_tc_tiling_on_sc=False),
        scratch_types=[
            pltpu.VMEM((chunk,), jnp.int32),
            pltpu.VMEM((ngrp, _GRP), jnp.int32),
            pltpu.VMEM((2, _GRP, HID), jnp.float32),
            pltpu.VMEM_SHARED((half + 8, HID), jnp.float32),
            pltpu.SemaphoreType.DMA,
            pltpu.SemaphoreType.DMA,
        ],
    )
    def k(msg_hbm, idx_hbm, zeros_hbm, out_hbm, idx_v, lidx_v, msg_v, acc_sh,
          sem0, sem1):
        c = lax.axis_index("c")
        s = lax.axis_index("s")
        half_i = jnp.int32(half)
        sems = [sem0, sem1]

        @pl.when(s < _NS - 1)
        def _():
            pltpu.sync_copy(zeros_hbm.at[pl.ds(0, rpt)],
                            acc_sh.at[pl.ds(s * rpt, rpt)])

        @pl.when(s == _NS - 1)
        def _():
            pltpu.sync_copy(zeros_hbm.at[pl.ds(0, last)],
                            acc_sh.at[pl.ds(s * rpt, last)])

        base = s * chunk
        pltpu.sync_copy(idx_hbm.at[pl.ds(base, chunk)], idx_v)
        for g in range(ngrp):
            for l in range(_GRP // 16):
                o = g * _GRP + l * 16
                v = idx_v[pl.ds(o, 16)] - c * half_i
                ok = (v >= 0) & (v < half_i)
                lidx_v[g, pl.ds(l * 16, 16)] = jnp.where(ok, v, half_i)

        plsc.subcore_barrier()

        loads = [None, None]
        loads[0] = pltpu.async_copy(
            msg_hbm.at[pl.ds(base, _GRP)], msg_v.at[0], sems[0])
        for g in range(ngrp):
            b = g % 2
            if g + 1 < ngrp:
                loads[1 - b] = pltpu.async_copy(
                    msg_hbm.at[pl.ds(base + (g + 1) * _GRP, _GRP)],
                    msg_v.at[1 - b], sems[1 - b])
            loads[b].wait()
            pltpu.sync_copy(msg_v.at[b], acc_sh.at[lidx_v.at[g]], add=True)

        plsc.subcore_barrier()

        @pl.when(s < _NS - 1)
        def _():
            pltpu.sync_copy(acc_sh.at[pl.ds(s * rpt, rpt)],
                            out_hbm.at[pl.ds(c * half + s * rpt, rpt)])

        @pl.when(s == _NS - 1)
        def _():
            pltpu.sync_copy(acc_sh.at[pl.ds(s * rpt, last)],
                            out_hbm.at[pl.ds(c * half + s * rpt, last)])

    return k(msg, idx, zeros)


def kernel(u_embedded, i_embedded, user_per_trans, item_per_trans, edges_t,
           u_t, i_t,
           wu_w1, wu_b1, wu_g1, wu_be1, wu_w2, wu_b2,
           wi_w1, wi_b1, wi_g1, wi_be1, wi_w2, wi_b2):
    e = edges_t.shape[0]
    n_users = u_embedded.shape[0]
    n_items = i_embedded.shape[0]
    eb = 1024
    quantum = _NS * _GRP
    e_pad = ((e + quantum - 1) // quantum) * quantum
    pad = e_pad - e
    nb = e_pad // eb

    up = jnp.pad(user_per_trans, (0, pad), constant_values=n_users)
    ip = jnp.pad(item_per_trans, (0, pad), constant_values=n_items)
    et = jnp.pad(edges_t, (0, pad))

    embg_i, rel_i_f = _gather_side(i_embedded, i_t, ip, et)
    rel_i = rel_i_f.reshape(nb, 1, eb)
    msg_item = _dense_side(rel_i, embg_i, wi_w1, wi_b1, wi_g1, wi_be1, wi_w2,
                           wi_b2, eb)

    embg_u, rel_u_f = _gather_side(u_embedded, u_t, up, et)
    rel_u = rel_u_f.reshape(nb, 1, eb)
    msg_user = _dense_side(rel_u, embg_u, wu_w1, wu_b1, wu_g1, wu_be1, wu_w2,
                           wu_b2, eb)

    zeros = jnp.zeros((-(-max(n_users, n_items) // (2 * _NS * 8)) * 8, HID),
                      jnp.float32)
    hlu = _scatter_add(msg_item, up, zeros, n_users)
    hli = _scatter_add(msg_user, ip, zeros, n_items)
    return (hlu, hli)
